# Initial kernel scaffold; baseline (speedup 1.0000x reference)
import jax
import jax.numpy as jnp
from jax.experimental import pallas as pl


def kernel(u_values, v_values):
    raise NotImplementedError("write your pallas kernel here")


# speedup vs baseline: 20.2003x; 20.2003x over previous
import functools
import jax
import jax.numpy as jnp
from jax.experimental import pallas as pl


def _tagged_keys(x, tag):
    b = jax.lax.bitcast_convert_type(x, jnp.uint32)
    k = jnp.where(b >> 31 != 0, ~b, b ^ jnp.uint32(0x80000000))
    return (k & jnp.uint32(0xFFFFFFFE)) | jnp.uint32(tag)


def _reduce_body(k_ref, o_ref):
    ri = jax.lax.broadcasted_iota(jnp.int32, (128, 128), 0)
    ci = jax.lax.broadcasted_iota(jnp.int32, (128, 128), 1)
    t128 = (ri <= ci).astype(jnp.float32)
    k = k_ref[0]
    sigma = 1.0 - 2.0 * (k & jnp.uint32(1)).astype(jnp.float32)
    b = jnp.where(k >> 31 != 0, k ^ jnp.uint32(0x80000000), ~k)
    x = jax.lax.bitcast_convert_type(b, jnp.float32)
    p1 = jax.lax.dot_general(sigma, t128, (((1,), (0,)), ((), ())),
                             preferred_element_type=jnp.float32)
    r1 = p1[:, 127:128]
    rows = r1.shape[0]
    inc = r1
    shift = 1
    while shift < rows:
        top = jnp.zeros((shift, 1), jnp.float32)
        inc = inc + jnp.concatenate([top, inc[:rows - shift]], axis=0)
        shift *= 2
    carry = inc - r1
    s = p1 + carry
    contrib = x * (1.0 - 2.0 * sigma * s)
    per_row = jnp.sum(contrib, axis=1, keepdims=True)
    o_ref[0] = jnp.sum(per_row, axis=0, keepdims=True)


def _pallas_reduce(keys):
    rows = keys.shape[0]
    k3 = keys.reshape(rows, keys.shape[1] // 128, 128)
    return pl.pallas_call(
        _reduce_body,
        grid=(rows,),
        in_specs=[pl.BlockSpec((1, k3.shape[1], 128), lambda i: (i, 0, 0))],
        out_specs=pl.BlockSpec((1, 1, 1), lambda i: (i, 0, 0)),
        out_shape=jax.ShapeDtypeStruct((rows, 1, 1), jnp.float32),
    )(k3)[:, 0, 0]


def kernel(u_values, v_values):
    n = u_values.shape[1]
    ku = _tagged_keys(u_values, 0)
    kv = _tagged_keys(v_values, 1)
    keys = jnp.sort(jnp.concatenate([ku, kv], axis=1), axis=1)
    t = _pallas_reduce(keys)
    dist = jnp.sqrt(jnp.maximum(t, 0.0)) / n
    return dist.sum() / u_values.shape[0]
